# Initial kernel scaffold; baseline (speedup 1.0000x reference)
import jax
import jax.numpy as jnp
from jax.experimental import pallas as pl


def kernel(coord, edge_index, edge_mask, Wn, bn, Ws, Wm, bg, We, be, W1, b1, W2, b2, W3, b3, W4, b4):
    raise NotImplementedError("write your pallas kernel here")


# speedup vs baseline: 2.8689x; 2.8689x over previous
import functools

import jax
import jax.numpy as jnp
from jax import lax
from jax.experimental import pallas as pl
from jax.experimental.pallas import tpu as pltpu
from jax.experimental.pallas import tpu_sc as plsc

N = 50000
E = 800000
D = 1024
ED = 64
HD = ED // 2
ES = 94
HID = 32

NC = 2
NS = 16

NPAD = 50176
HALF = NPAD // 2
AROWS = 25600
DUMP = HALF
EPAD = 802816
EROWS = EPAD // 128
TROWS = EROWS // NS
SUP = 8
NSUP = TROWS // SUP
ESP = 128
MPAD = D * ESP
TSEL = MPAD // (NC * NS)

_mesh = plsc.VectorSubcoreMesh(
    core_axis_name="c", subcore_axis_name="s", num_cores=NC, num_subcores=NS)
_sc_params = pltpu.CompilerParams(use_tc_tiling_on_sc=False)


def _leaky(x):
  return jnp.where(x >= 0, x, 0.01 * x)


@functools.partial(
    pl.kernel,
    out_type=(
        jax.ShapeDtypeStruct((NPAD, HD), jnp.float32),
        jax.ShapeDtypeStruct((NPAD, HD), jnp.float32),
    ),
    mesh=_mesh,
    scratch_types=[
        pltpu.VMEM((SUP, 128), jnp.int32),
        pltpu.VMEM((SUP, 128), jnp.int32),
        pltpu.VMEM((128, HD), jnp.float32),
        pltpu.VMEM((512, HD), jnp.float32),
        pltpu.VMEM_SHARED((AROWS, HD), jnp.float32),
        pltpu.SemaphoreType.DMA,
    ],
    compiler_params=_sc_params,
)
def _segsum(nflo, nfhi, src2d, dlocs, zeros_hbm, msg0, msg1,
            src_v, didx_v, rows, buf, acc, sem):
  c = lax.axis_index("c")
  s = lax.axis_index("s")
  base = c * HALF

  for p in range(2):
    nf_t = nflo if p == 0 else nfhi
    out_t = msg0 if p == 0 else msg1

    pltpu.sync_copy(zeros_hbm, buf)
    for q in range(3):
      pltpu.sync_copy(buf, acc.at[pl.ds(s * 1600 + q * 512, 512)])
    pltpu.sync_copy(buf.at[pl.ds(0, 64)], acc.at[pl.ds(s * 1600 + 1536, 64)])
    plsc.subcore_barrier()

    def sup_body(it, _):
      r0 = s * TROWS + it * SUP
      pltpu.sync_copy(src2d.at[pl.ds(r0, SUP)], src_v)
      pltpu.sync_copy(dlocs.at[c, pl.ds(r0, SUP)], didx_v)
      for j in range(SUP):
        pltpu.async_copy(nf_t.at[src_v.at[j]], rows, sem).wait()
        pltpu.sync_copy(rows, acc.at[didx_v.at[j]], add=True)
      return 0

    lax.fori_loop(0, NSUP, sup_body, 0)
    plsc.subcore_barrier()

    for q in range(3):
      pltpu.sync_copy(acc.at[pl.ds(s * 1568 + q * 512, 512)], buf)
      pltpu.sync_copy(buf, out_t.at[pl.ds(base + s * 1568 + q * 512, 512)])
    pltpu.sync_copy(acc.at[pl.ds(s * 1568 + 1536, 32)], buf.at[pl.ds(0, 32)])
    pltpu.sync_copy(buf.at[pl.ds(0, 32)],
                    out_t.at[pl.ds(base + s * 1568 + 1536, 32)])
    plsc.subcore_barrier()


@functools.partial(
    pl.kernel,
    out_type=tuple(
        jax.ShapeDtypeStruct((MPAD, HD), jnp.float32) for _ in range(4)),
    mesh=_mesh,
    scratch_types=[
        pltpu.VMEM((TSEL,), jnp.int32),
        pltpu.VMEM((128,), jnp.int32),
        pltpu.VMEM((128,), jnp.int32),
        pltpu.VMEM((128, HD), jnp.float32),
        pltpu.SemaphoreType.DMA,
    ],
    compiler_params=_sc_params,
)
def _selgather(nflo, nfhi, srce_hbm, dste_hbm, mask_hbm,
               slo, shi, dlo, dhi, mask_st, sid, did, rows, sem):
  c = lax.axis_index("c")
  s = lax.axis_index("s")
  wid = s * NC + c
  tbase = wid * TSEL
  pltpu.sync_copy(mask_hbm.at[pl.ds(tbase, TSEL)], mask_st)

  def chunk_body(j, _):
    msl = mask_st.at[pl.ds(j * 128, 128)]
    pltpu.async_copy(srce_hbm.at[msl], sid, sem).wait()
    pltpu.async_copy(dste_hbm.at[msl], did, sem).wait()
    for tbl, idx, out in ((nflo, sid, slo), (nfhi, sid, shi),
                          (nflo, did, dlo), (nfhi, did, dhi)):
      pltpu.async_copy(tbl.at[idx], rows, sem).wait()
      pltpu.sync_copy(rows, out.at[pl.ds(tbase + j * 128, 128)])
    return 0

  lax.fori_loop(0, TSEL // 128, chunk_body, 0)


_RB = 512
_NBLK = NPAD // _RB
_EB = 64
_EBLK = EROWS // _EB


def _dloc_body(dst_ref, out_ref):
  d = dst_ref[...]
  out_ref[0] = jnp.where(d < HALF, d, DUMP)
  out_ref[1] = jnp.where(d >= HALF, d - HALF, DUMP)


def _tc_dloc(dst2d):
  return pl.pallas_call(
      _dloc_body,
      grid=(_EBLK,),
      in_specs=[pl.BlockSpec((_EB, 128), lambda i: (i, 0))],
      out_specs=pl.BlockSpec((2, _EB, 128), lambda i: (0, i, 0)),
      out_shape=jax.ShapeDtypeStruct((2, EROWS, 128), jnp.int32),
  )(dst2d)


def _embed_body(coord_ref, wn_ref, bn_ref, lo_ref, hi_ref):
  r = (jnp.dot(coord_ref[...], wn_ref[...],
               preferred_element_type=jnp.float32) + bn_ref[...])
  lo_ref[...] = r[:, :HD]
  hi_ref[...] = r[:, HD:]


def _tc_embed(coordp, Wn, bn2):
  return pl.pallas_call(
      _embed_body,
      grid=(_NBLK,),
      in_specs=[
          pl.BlockSpec((_RB, 2), lambda i: (i, 0)),
          pl.BlockSpec((2, ED), lambda i: (0, 0)),
          pl.BlockSpec((1, ED), lambda i: (0, 0)),
      ],
      out_specs=[
          pl.BlockSpec((_RB, HD), lambda i: (i, 0)),
          pl.BlockSpec((_RB, HD), lambda i: (i, 0)),
      ],
      out_shape=[
          jax.ShapeDtypeStruct((NPAD, HD), jnp.float32),
          jax.ShapeDtypeStruct((NPAD, HD), jnp.float32),
      ],
  )(coordp, Wn, bn2)


def _dense_body(lo_ref, hi_ref, m0_ref, m1_ref, ws_ref, wm_ref,
                bg_ref, lo_out, hi_out):
  nf = jnp.concatenate([lo_ref[...], hi_ref[...]], axis=1)
  msg = jnp.concatenate([m0_ref[...], m1_ref[...]], axis=1)
  h = (jnp.dot(nf, ws_ref[...], preferred_element_type=jnp.float32)
       + jnp.dot(msg, wm_ref[...], preferred_element_type=jnp.float32)
       + bg_ref[...])
  r = nf + _leaky(h)
  lo_out[...] = r[:, :HD]
  hi_out[...] = r[:, HD:]


def _tc_dense(lo, hi, m0, m1, Wsl, Wml, bgl2):
  return pl.pallas_call(
      _dense_body,
      grid=(_NBLK,),
      in_specs=[
          pl.BlockSpec((_RB, HD), lambda i: (i, 0)),
          pl.BlockSpec((_RB, HD), lambda i: (i, 0)),
          pl.BlockSpec((_RB, HD), lambda i: (i, 0)),
          pl.BlockSpec((_RB, HD), lambda i: (i, 0)),
          pl.BlockSpec((ED, ED), lambda i: (0, 0)),
          pl.BlockSpec((ED, ED), lambda i: (0, 0)),
          pl.BlockSpec((1, ED), lambda i: (0, 0)),
      ],
      out_specs=[
          pl.BlockSpec((_RB, HD), lambda i: (i, 0)),
          pl.BlockSpec((_RB, HD), lambda i: (i, 0)),
      ],
      out_shape=[
          jax.ShapeDtypeStruct((NPAD, HD), jnp.float32),
          jax.ShapeDtypeStruct((NPAD, HD), jnp.float32),
      ],
  )(lo, hi, m0, m1, Wsl, Wml, bgl2)


_SB = 8192
_SBLK = MPAD // _SB


def _mlpa_body(slo_ref, shi_ref, dlo_ref, dhi_ref, we_ref, be_ref,
               w1_ref, b1_ref, w2_ref, b2_ref, out_ref):
  x = jnp.concatenate(
      [slo_ref[...], shi_ref[...], dlo_ref[...], dhi_ref[...]], axis=1)
  ef = (jnp.dot(x, we_ref[...], preferred_element_type=jnp.float32)
        + be_ref[...])
  hid = _leaky(
      jnp.dot(ef, w1_ref[...], preferred_element_type=jnp.float32)
      + b1_ref[...])
  out_ref[...] = (
      jnp.dot(hid, w2_ref[...], preferred_element_type=jnp.float32)
      + b2_ref[...])


def _tc_mlpa(slo, shi, dlo, dhi, We, be2, W1, b12, W2, b22):
  return pl.pallas_call(
      _mlpa_body,
      grid=(_SBLK,),
      in_specs=[
          pl.BlockSpec((_SB, HD), lambda i: (i, 0)),
          pl.BlockSpec((_SB, HD), lambda i: (i, 0)),
          pl.BlockSpec((_SB, HD), lambda i: (i, 0)),
          pl.BlockSpec((_SB, HD), lambda i: (i, 0)),
          pl.BlockSpec((2 * ED, ED), lambda i: (0, 0)),
          pl.BlockSpec((1, ED), lambda i: (0, 0)),
          pl.BlockSpec((ED, HID), lambda i: (0, 0)),
          pl.BlockSpec((1, HID), lambda i: (0, 0)),
          pl.BlockSpec((HID, 1), lambda i: (0, 0)),
          pl.BlockSpec((1, 1), lambda i: (0, 0)),
      ],
      out_specs=pl.BlockSpec((_SB, 1), lambda i: (i, 0)),
      out_shape=jax.ShapeDtypeStruct((MPAD, 1), jnp.float32),
  )(slo, shi, dlo, dhi, We, be2, W1, b12, W2, b22)


def _mlpb_body(x_ref, w3_ref, b3_ref, w4_ref, b4_ref, out_ref):
  t = _leaky(
      jnp.dot(x_ref[...], w3_ref[...], preferred_element_type=jnp.float32)
      + b3_ref[...])
  out_ref[...] = (
      jnp.dot(t, w4_ref[...], preferred_element_type=jnp.float32)
      + b4_ref[...])


def _tc_mlpb(x2d, W3p, b32, W4, b42):
  return pl.pallas_call(
      _mlpb_body,
      grid=(1,),
      in_specs=[
          pl.BlockSpec((D, ESP), lambda i: (0, 0)),
          pl.BlockSpec((ESP, HID), lambda i: (0, 0)),
          pl.BlockSpec((1, HID), lambda i: (0, 0)),
          pl.BlockSpec((HID, 1), lambda i: (0, 0)),
          pl.BlockSpec((1, 1), lambda i: (0, 0)),
      ],
      out_specs=pl.BlockSpec((D, 1), lambda i: (0, 0)),
      out_shape=jax.ShapeDtypeStruct((D, 1), jnp.float32),
  )(x2d, W3p, b32, W4, b42)


def kernel(coord, edge_index, edge_mask, Wn, bn, Ws, Wm, bg, We, be,
           W1, b1, W2, b2, W3, b3, W4, b4):
  f32 = jnp.float32
  src = edge_index[0]
  dst = edge_index[1]

  coordp = jnp.concatenate([coord, jnp.zeros((NPAD - N, 2), f32)], axis=0)
  srcp = jnp.concatenate([src, jnp.zeros((EPAD - E,), jnp.int32)], axis=0)
  dstp = jnp.concatenate(
      [dst, jnp.full((EPAD - E,), NPAD, jnp.int32)], axis=0)
  src2d = srcp.reshape(EROWS, 128)
  dst2d = dstp.reshape(EROWS, 128)
  maskp = jnp.concatenate(
      [edge_mask, jnp.zeros((D, ESP - ES), jnp.int32)], axis=1).reshape(MPAD)
  zeros512 = jnp.zeros((512, HD), f32)
  W3p = jnp.concatenate([W3, jnp.zeros((ESP - ES, HID), f32)], axis=0)
  bn2 = bn.reshape(1, ED)
  be2 = be.reshape(1, ED)
  b12 = b1.reshape(1, HID)
  b22 = b2.reshape(1, 1)
  b32 = b3.reshape(1, HID)
  b42 = b4.reshape(1, 1)

  dlocs = _tc_dloc(dst2d)
  lo, hi = _tc_embed(coordp, Wn, bn2)
  for l in range(3):
    m0, m1 = _segsum(lo, hi, src2d, dlocs, zeros512)
    lo, hi = _tc_dense(lo, hi, m0, m1, Ws[l], Wm[l], bg[l].reshape(1, ED))
  slo, shi, dlo, dhi = _selgather(lo, hi, srcp, dstp, maskp)
  x = _tc_mlpa(slo, shi, dlo, dhi, We, be2, W1, b12, W2, b22)
  x2d = x.reshape(D, ESP)
  out = _tc_mlpb(x2d, W3p, b32, W4, b42)
  return out.reshape(D)
